# baseline (speedup 1.0000x reference)
import functools

import jax
import jax.numpy as jnp
import numpy as np
from jax import lax
from jax.experimental import pallas as pl
from jax.experimental.pallas import tpu as pltpu
from jax.experimental.pallas import tpu_sc as plsc

_NUM_WORKERS = 32
_CHUNK = 640


def kernel(inp_data, emb_table):
    B, T = inp_data.shape
    V, E = emb_table.shape
    N = B * T
    n_per_w = N // _NUM_WORKERS
    n_chunks = n_per_w // _CHUNK
    b_chunk = _CHUNK // T

    jj = np.arange(_CHUNK)
    pos_pat = jnp.asarray(128 * (jj // T) + jj % T, dtype=jnp.int32)

    idx1 = jnp.pad(inp_data.astype(jnp.int32),
                   ((0, 0), (0, 128 - T))).reshape(B * 128)

    mesh = plsc.VectorSubcoreMesh(core_axis_name="c", subcore_axis_name="s")

    @functools.partial(
        pl.kernel,
        mesh=mesh,
        compiler_params=pltpu.CompilerParams(use_tc_tiling_on_sc=False,
                                             needs_layout_passes=False),
        out_type=jax.ShapeDtypeStruct((N, 2 * E), emb_table.dtype),
        scratch_types=[
            pltpu.VMEM((_CHUNK,), jnp.int32),
            [pltpu.VMEM((128 * b_chunk,), jnp.int32)] * 2,
            [pltpu.VMEM((_CHUNK,), jnp.int32)] * 2,
            [pltpu.VMEM((_CHUNK, E), emb_table.dtype)] * 2,
            [pltpu.SemaphoreType.DMA] * 2,
        ],
    )
    def gather_kernel(tbl_hbm, idx_hbm, pos_hbm, z_hbm, pos_v, idx_v, iv_v,
                      rows_v, sem):
        wid = lax.axis_index("s") * 2 + lax.axis_index("c")
        nbase = wid * n_per_w
        pltpu.sync_copy(pos_hbm, pos_v)

        def stage(c, buf):
            noff = nbase + c * _CHUNK
            boff = noff // T
            pltpu.sync_copy(idx_hbm.at[pl.ds(128 * boff, 128 * b_chunk)],
                            idx_v[buf])
            for v in range(_CHUNK // 16):
                p = pos_v[pl.ds(16 * v, 16)]
                iv_v[buf][pl.ds(16 * v, 16)] = (
                    plsc.load_gather(idx_v[buf], [p]))
            return pltpu.async_copy(tbl_hbm.at[iv_v[buf]], rows_v[buf],
                                    sem[buf])

        def drain(c, buf, cp):
            noff = nbase + c * _CHUNK
            cp.wait()
            pltpu.sync_copy(rows_v[buf],
                            z_hbm.at[pl.ds(noff, _CHUNK), pl.ds(0, E)])

        cp = stage(0, 0)
        for c in range(1, n_chunks):
            cn = stage(c, c % 2)
            drain(c - 1, (c - 1) % 2, cp)
            cp = cn
        drain(n_chunks - 1, (n_chunks - 1) % 2, cp)

    z = gather_kernel(emb_table, idx1, pos_pat)
    return z[:, :E].reshape(B, T, E)


# speedup vs baseline: 1.3977x; 1.3977x over previous
import functools

import jax
import jax.numpy as jnp
import numpy as np
from jax import lax
from jax.experimental import pallas as pl
from jax.experimental.pallas import tpu as pltpu
from jax.experimental.pallas import tpu_sc as plsc

_NUM_WORKERS = 32
_ZCHUNK = 320
_GB = 64


def kernel(inp_data, emb_table):
    B, T = inp_data.shape
    V, E = emb_table.shape
    N = B * T
    CB = (T * E) // 128
    RB = B // 8
    NZ = N // 2
    z_per_w = NZ // _NUM_WORKERS
    n_chunks = z_per_w // _ZCHUNK
    b_chunk = 2 * _ZCHUNK // T

    jz = np.arange(_ZCHUNK)
    r = jz % 8
    t = (jz // 8) % CB
    tb = jz // (8 * CB)
    prow_pat = jnp.asarray(8 * tb + r, dtype=jnp.int32)
    pcol_pat = jnp.asarray(2 * t, dtype=jnp.int32)

    idxf = jax.lax.bitcast_convert_type(inp_data.astype(jnp.int32),
                                        jnp.float32)

    mesh = plsc.VectorSubcoreMesh(core_axis_name="c", subcore_axis_name="s")

    @functools.partial(
        pl.kernel,
        mesh=mesh,
        compiler_params=pltpu.CompilerParams(use_tc_tiling_on_sc=False,
                                             needs_layout_passes=False),
        out_type=jax.ShapeDtypeStruct((NZ, 2 * E), emb_table.dtype),
        scratch_types=[
            pltpu.VMEM((_ZCHUNK,), jnp.int32),
            pltpu.VMEM((_ZCHUNK,), jnp.int32),
            [pltpu.VMEM((b_chunk, T), jnp.float32)] * 2,
            [pltpu.VMEM((_ZCHUNK,), jnp.int32)] * 2,
            [pltpu.VMEM((_ZCHUNK,), jnp.int32)] * 2,
            [pltpu.VMEM((_ZCHUNK, E), emb_table.dtype)] * 2,
            [pltpu.VMEM((_ZCHUNK, E), emb_table.dtype)] * 2,
            [pltpu.SemaphoreType.DMA] * 2,
        ],
    )
    def gather_kernel(tbl_hbm, idx_hbm, prow_hbm, pcol_hbm, z_hbm,
                      prow_v, pcol_v, idx_v, ie_v, io_v, re_v, ro_v, sem):
        wid = lax.axis_index("s") * 2 + lax.axis_index("c")
        zbase = wid * z_per_w
        pltpu.sync_copy(prow_hbm, prow_v)
        pltpu.sync_copy(pcol_hbm, pcol_v)

        def stage(c, buf):
            zoff = zbase + c * _ZCHUNK
            boff = 2 * zoff // T
            pltpu.sync_copy(idx_hbm.at[pl.ds(boff, b_chunk)], idx_v[buf])
            for v in range(_ZCHUNK // 16):
                pr = prow_v[pl.ds(16 * v, 16)]
                pc = pcol_v[pl.ds(16 * v, 16)]
                ie_v[buf][pl.ds(16 * v, 16)] = plsc.bitcast(
                    plsc.load_gather(idx_v[buf], [pr, pc]), jnp.int32)
                io_v[buf][pl.ds(16 * v, 16)] = plsc.bitcast(
                    plsc.load_gather(idx_v[buf], [pr, pc + 1]), jnp.int32)
            ce = pltpu.async_copy(tbl_hbm.at[ie_v[buf]], re_v[buf], sem[buf])
            co = pltpu.async_copy(tbl_hbm.at[io_v[buf]], ro_v[buf], sem[buf])
            return ce, co

        def drain(c, buf, ce, co):
            zoff = zbase + c * _ZCHUNK
            ce.wait()
            co.wait()
            pltpu.sync_copy(re_v[buf],
                            z_hbm.at[pl.ds(zoff, _ZCHUNK), pl.ds(0, E)])
            pltpu.sync_copy(ro_v[buf],
                            z_hbm.at[pl.ds(zoff, _ZCHUNK), pl.ds(E, E)])

        cp = stage(0, 0)
        for c in range(1, n_chunks):
            cn = stage(c, c % 2)
            drain(c - 1, (c - 1) % 2, *cp)
            cp = cn
        drain(n_chunks - 1, (n_chunks - 1) % 2, *cp)

    z = gather_kernel(emb_table, idxf, prow_pat, pcol_pat)

    z4 = z.reshape(RB, CB, 8, 2 * E)

    def untile_body(x_ref, o_ref):
        for tc in range(CB):
            o_ref[:, 128 * tc:128 * (tc + 1)] = (
                x_ref[:, tc].reshape(_GB * 8, 2 * E))

    y = pl.pallas_call(
        untile_body,
        grid=(RB // _GB,),
        in_specs=[pl.BlockSpec((_GB, CB, 8, 2 * E), lambda i: (i, 0, 0, 0))],
        out_specs=pl.BlockSpec((_GB * 8, T * E), lambda i: (i, 0)),
        out_shape=jax.ShapeDtypeStruct((B, T * E), emb_table.dtype),
    )(z4)
    return y.reshape(B, T, E)
